# speedup vs baseline: 1.4687x; 1.4687x over previous
import functools

import jax
import jax.numpy as jnp
from jax import lax
from jax.experimental import pallas as pl
from jax.experimental.pallas import tpu as pltpu
from jax.experimental.pallas import tpu_sc as plsc


def _make_embed_kernel(N, D, NC, NS, chunk):
    NW = NC * NS
    per_w = N // NW
    n_chunks = per_w // chunk
    mesh = plsc.VectorSubcoreMesh(core_axis_name="c", subcore_axis_name="s")

    @functools.partial(
        pl.kernel,
        out_type=jax.ShapeDtypeStruct((N, D), jnp.float32),
        mesh=mesh,
        scratch_types=[
            pltpu.VMEM((n_chunks, chunk), jnp.int32),
            pltpu.VMEM((chunk, D), jnp.float32),
            pltpu.VMEM((chunk, D), jnp.float32),
            pltpu.SemaphoreType.DMA,
            pltpu.SemaphoreType.DMA,
        ],
    )
    def k(tokens_hbm, table_hbm, out_hbm, idx_v, rows0, rows1, sem0, sem1):
        wid = lax.axis_index("s") * NC + lax.axis_index("c")
        base = wid * per_w
        pltpu.sync_copy(tokens_hbm.at[wid], idx_v)

        bufs = (rows0, rows1)
        sems = (sem0, sem1)
        futs = [None, None]
        futs[0] = pltpu.async_copy(table_hbm.at[idx_v.at[0]], rows0, sem0)
        for j in range(n_chunks):
            if j + 1 < n_chunks:
                futs[(j + 1) % 2] = pltpu.async_copy(
                    table_hbm.at[idx_v.at[j + 1]], bufs[(j + 1) % 2],
                    sems[(j + 1) % 2])
            futs[j % 2].wait()
            pltpu.sync_copy(bufs[j % 2],
                            out_hbm.at[pl.ds(base + j * chunk, chunk)])

    return k


def kernel(tokens, W_E):
    B, S = tokens.shape
    V, D = W_E.shape
    N = B * S
    info = plsc.get_sparse_core_info()
    NC, NS = info.num_cores, info.num_subcores
    NW = NC * NS
    chunk = 64
    per_w = N // NW
    tokens_grp = tokens.reshape(NW, per_w // chunk, chunk).astype(jnp.int32)
    out = _make_embed_kernel(N, D, NC, NS, chunk)(tokens_grp, W_E)
    return out.reshape(B, S, D)


# speedup vs baseline: 1.5156x; 1.0319x over previous
import functools

import jax
import jax.numpy as jnp
from jax import lax
from jax.experimental import pallas as pl
from jax.experimental.pallas import tpu as pltpu
from jax.experimental.pallas import tpu_sc as plsc


def _make_embed_kernel(N, D, NC, NS, chunk, nbuf):
    NW = NC * NS
    per_w = N // NW
    n_chunks = per_w // chunk
    prime = min(nbuf - 1, n_chunks)
    mesh = plsc.VectorSubcoreMesh(core_axis_name="c", subcore_axis_name="s")

    @functools.partial(
        pl.kernel,
        out_type=jax.ShapeDtypeStruct((N, D), jnp.float32),
        mesh=mesh,
        scratch_types=(
            [pltpu.VMEM((n_chunks, chunk), jnp.int32)]
            + [pltpu.VMEM((chunk, D), jnp.float32) for _ in range(nbuf)]
            + [pltpu.SemaphoreType.DMA for _ in range(2 * nbuf)]
        ),
    )
    def k(tokens_hbm, table_hbm, out_hbm, idx_v, *rest):
        bufs = rest[:nbuf]
        gsems = rest[nbuf:2 * nbuf]
        wsems = rest[2 * nbuf:3 * nbuf]
        wid = lax.axis_index("s") * NC + lax.axis_index("c")
        base = wid * per_w
        pltpu.sync_copy(tokens_hbm.at[wid], idx_v)

        gfut = [None] * nbuf
        wfut = [None] * nbuf
        for j in range(prime):
            gfut[j] = pltpu.async_copy(
                table_hbm.at[idx_v.at[j]], bufs[j], gsems[j])
        for j in range(n_chunks):
            b = j % nbuf
            nxt = j + prime
            if nxt < n_chunks:
                b2 = nxt % nbuf
                if wfut[b2] is not None:
                    wfut[b2].wait()
                gfut[b2] = pltpu.async_copy(
                    table_hbm.at[idx_v.at[nxt]], bufs[b2], gsems[b2])
            gfut[b].wait()
            wfut[b] = pltpu.async_copy(
                bufs[b], out_hbm.at[pl.ds(base + j * chunk, chunk)], wsems[b])
        for j in range(max(0, n_chunks - nbuf), n_chunks):
            wfut[j % nbuf].wait()

    return k


def kernel(tokens, W_E):
    B, S = tokens.shape
    V, D = W_E.shape
    N = B * S
    info = plsc.get_sparse_core_info()
    NC, NS = info.num_cores, info.num_subcores
    NW = NC * NS
    chunk, nbuf = 32, 4
    per_w = N // NW
    tokens_grp = tokens.reshape(NW, per_w // chunk, chunk).astype(jnp.int32)
    out = _make_embed_kernel(N, D, NC, NS, chunk, nbuf)(tokens_grp, W_E)
    return out.reshape(B, S, D)


# speedup vs baseline: 1.5335x; 1.0119x over previous
import functools

import jax
import jax.numpy as jnp
from jax import lax
from jax.experimental import pallas as pl
from jax.experimental.pallas import tpu as pltpu
from jax.experimental.pallas import tpu_sc as plsc


def _make_embed_kernel(N, D, NC, NS, chunk, nbuf):
    NW = NC * NS
    per_w = N // NW
    n_chunks = per_w // chunk
    prime = min(nbuf - 1, n_chunks)
    mesh = plsc.VectorSubcoreMesh(core_axis_name="c", subcore_axis_name="s")

    @functools.partial(
        pl.kernel,
        out_type=jax.ShapeDtypeStruct((N, D), jnp.float32),
        mesh=mesh,
        scratch_types=(
            [pltpu.VMEM((per_w,), jnp.int32)]
            + [pltpu.VMEM((chunk, D), jnp.float32) for _ in range(nbuf)]
            + [pltpu.SemaphoreType.DMA for _ in range(2 * nbuf)]
        ),
    )
    def k(tokens_hbm, table_hbm, out_hbm, idx_v, *rest):
        bufs = rest[:nbuf]
        gsems = rest[nbuf:2 * nbuf]
        wsems = rest[2 * nbuf:3 * nbuf]
        wid = lax.axis_index("s") * NC + lax.axis_index("c")
        base = wid * per_w
        S = tokens_hbm.shape[1]
        w_per_row = S // per_w
        pltpu.sync_copy(
            tokens_hbm.at[wid // w_per_row,
                          pl.ds((wid % w_per_row) * per_w, per_w)], idx_v)

        def gidx(j):
            return idx_v.at[pl.ds(j * chunk, chunk)]

        gfut = [None] * nbuf
        wfut = [None] * nbuf
        for j in range(prime):
            gfut[j] = pltpu.async_copy(
                table_hbm.at[gidx(j)], bufs[j], gsems[j])
        for j in range(n_chunks):
            b = j % nbuf
            nxt = j + prime
            if nxt < n_chunks:
                b2 = nxt % nbuf
                if wfut[b2] is not None:
                    wfut[b2].wait()
                gfut[b2] = pltpu.async_copy(
                    table_hbm.at[gidx(nxt)], bufs[b2], gsems[b2])
            gfut[b].wait()
            wfut[b] = pltpu.async_copy(
                bufs[b], out_hbm.at[pl.ds(base + j * chunk, chunk)], wsems[b])
        for j in range(max(0, n_chunks - nbuf), n_chunks):
            wfut[j % nbuf].wait()

    return k


def kernel(tokens, W_E):
    B, S = tokens.shape
    V, D = W_E.shape
    N = B * S
    info = plsc.get_sparse_core_info()
    NC, NS = info.num_cores, info.num_subcores
    NW = NC * NS
    chunk, nbuf = 32, 5
    out = _make_embed_kernel(N, D, NC, NS, chunk, nbuf)(tokens, W_E)
    return out.reshape(B, S, D)
